# baseline (speedup 1.0000x reference)
import functools

import jax
import jax.numpy as jnp
from jax import lax
from jax.experimental import pallas as pl
from jax.experimental.pallas import tpu as pltpu
from jax.experimental.pallas import tpu_sc as plsc

NC = 2
NS = 16
NW = NC * NS
CH = 128


def _node_linear(x, w, b):
    n, d = x.shape
    br = 1000
    grid = (n // br,)

    def body(x_ref, w_ref, b_ref, o_ref):
        o_ref[...] = lax.dot_general(
            x_ref[...], w_ref[...], (((1,), (1,)), ((), ())),
            preferred_element_type=jnp.float32) + b_ref[...]

    return pl.pallas_call(
        body,
        grid=grid,
        in_specs=[
            pl.BlockSpec((br, d), lambda i: (i, 0)),
            pl.BlockSpec((d, d), lambda i: (0, 0)),
            pl.BlockSpec((1, d), lambda i: (0, 0)),
        ],
        out_specs=pl.BlockSpec((br, d), lambda i: (i, 0)),
        out_shape=jax.ShapeDtypeStruct((n, d), jnp.float32),
    )(x, w, b.reshape(1, d))


def _combine(p0, p1):
    n, d = p0.shape
    br = 1000
    grid = (n // br,)

    def body(a_ref, b_ref, o_ref):
        o_ref[...] = a_ref[...] + b_ref[...]

    return pl.pallas_call(
        body,
        grid=grid,
        in_specs=[
            pl.BlockSpec((br, d), lambda i: (i, 0)),
            pl.BlockSpec((br, d), lambda i: (i, 0)),
        ],
        out_specs=pl.BlockSpec((br, d), lambda i: (i, 0)),
        out_shape=jax.ShapeDtypeStruct((n, d), jnp.float32),
    )(p0, p1)


NBUF = 2


def _scatter_partials(h, n0r, n1r, zrows, n_pad, steps):
    _, d = h.shape
    rps = n_pad // NS
    mesh = plsc.VectorSubcoreMesh(core_axis_name="c", subcore_axis_name="s")
    assert steps % NBUF == 0

    @functools.partial(
        pl.kernel,
        out_type=jax.ShapeDtypeStruct((NC, n_pad, d), jnp.float32),
        mesh=mesh,
        scratch_types=[
            pltpu.VMEM((CH,), jnp.int32),
            pltpu.VMEM((CH,), jnp.int32),
            pltpu.VMEM((CH, d), jnp.float32),
            pltpu.VMEM_SHARED((n_pad, d), jnp.float32),
            pltpu.SemaphoreType.DMA,
        ],
    )
    def k(h_hbm, n0_hbm, n1_hbm, z_hbm, out_hbm, idx0, idx1, rows, acc, sem):
        c = lax.axis_index("c")
        s = lax.axis_index("s")
        wid = c * NS + s
        pltpu.sync_copy(z_hbm, acc.at[pl.ds(s * rps, rps)])
        plsc.subcore_barrier()

        def step(i, carry):
            rb = wid * steps + i
            pltpu.sync_copy(n1_hbm.at[rb], idx1)
            pltpu.sync_copy(n0_hbm.at[rb], idx0)
            pltpu.async_copy(h_hbm.at[idx1], rows, sem).wait()
            pltpu.sync_copy(rows, acc.at[idx0], add=True)
            return carry

        lax.fori_loop(0, steps, step, 0)
        plsc.subcore_barrier()
        pltpu.sync_copy(acc.at[pl.ds(s * rps, rps)],
                        out_hbm.at[c].at[pl.ds(s * rps, rps)])

    return k(h, n0r, n1r, zrows)


def kernel(node_attributes, edge_attributes, W_node, b_node, W_edge, b_edge,
           W_att, b_att, edge_node_indices):
    n, d = node_attributes.shape
    e = edge_node_indices.shape[1]

    steps = -(-e // (NW * CH * 8)) * 8
    e_pad = NW * steps * CH
    n_pad = -(-(n + 1) // (NS * 8)) * (NS * 8)

    n0 = edge_node_indices[0]
    n1 = edge_node_indices[1]
    pad = e_pad - e
    n0r = jnp.concatenate([n0, jnp.full((pad,), n, jnp.int32)]).reshape(-1, CH)
    n1r = jnp.concatenate([n1, jnp.zeros((pad,), jnp.int32)]).reshape(-1, CH)
    zrows = jnp.zeros((n_pad // NS, d), jnp.float32)

    h = _node_linear(node_attributes, W_node, b_node)
    partials = _scatter_partials(h, n0r, n1r, zrows, n_pad, steps)
    return _combine(partials[0, :n], partials[1, :n])


# speedup vs baseline: 1.4610x; 1.4610x over previous
import functools

import jax
import jax.numpy as jnp
from jax import lax
from jax.experimental import pallas as pl
from jax.experimental.pallas import tpu as pltpu
from jax.experimental.pallas import tpu_sc as plsc

NC = 2
NS = 16
NW = NC * NS
CH = 128


def _node_linear(x, w, b):
    n, d = x.shape
    br = 1000
    grid = (n // br,)

    def body(x_ref, w_ref, b_ref, o_ref):
        o_ref[...] = lax.dot_general(
            x_ref[...], w_ref[...], (((1,), (1,)), ((), ())),
            preferred_element_type=jnp.float32) + b_ref[...]

    return pl.pallas_call(
        body,
        grid=grid,
        in_specs=[
            pl.BlockSpec((br, d), lambda i: (i, 0)),
            pl.BlockSpec((d, d), lambda i: (0, 0)),
            pl.BlockSpec((1, d), lambda i: (0, 0)),
        ],
        out_specs=pl.BlockSpec((br, d), lambda i: (i, 0)),
        out_shape=jax.ShapeDtypeStruct((n, d), jnp.float32),
    )(x, w, b.reshape(1, d))


def _combine(p0, p1):
    n, d = p0.shape
    br = 1000
    grid = (n // br,)

    def body(a_ref, b_ref, o_ref):
        o_ref[...] = a_ref[...] + b_ref[...]

    return pl.pallas_call(
        body,
        grid=grid,
        in_specs=[
            pl.BlockSpec((br, d), lambda i: (i, 0)),
            pl.BlockSpec((br, d), lambda i: (i, 0)),
        ],
        out_specs=pl.BlockSpec((br, d), lambda i: (i, 0)),
        out_shape=jax.ShapeDtypeStruct((n, d), jnp.float32),
    )(p0, p1)


NBUF = 2


def _scatter_partials(h, n0r, n1r, zrows, n_pad, steps):
    _, d = h.shape
    rps = n_pad // NS
    mesh = plsc.VectorSubcoreMesh(core_axis_name="c", subcore_axis_name="s")

    @functools.partial(
        pl.kernel,
        out_type=jax.ShapeDtypeStruct((NC, n_pad, d), jnp.float32),
        mesh=mesh,
        scratch_types=[
            pltpu.VMEM((CH,), jnp.int32),
            pltpu.VMEM((CH,), jnp.int32),
            pltpu.VMEM((CH, d), jnp.float32),
            pltpu.VMEM_SHARED((n_pad, d), jnp.float32),
            pltpu.SemaphoreType.DMA,
        ],
    )
    def k(h_hbm, n0_hbm, n1_hbm, z_hbm, out_hbm, idx0, idx1, rows, acc, sem):
        c = lax.axis_index("c")
        s = lax.axis_index("s")
        wid = c * NS + s
        pltpu.sync_copy(z_hbm, acc.at[pl.ds(s * rps, rps)])
        plsc.subcore_barrier()

        def step(i, carry):
            rb = wid * steps + i
            pltpu.sync_copy(n1_hbm.at[rb], idx1)
            pltpu.sync_copy(n0_hbm.at[rb], idx0)
            pltpu.async_copy(h_hbm.at[idx1], rows, sem).wait()
            pltpu.sync_copy(rows, acc.at[idx0], add=True)
            return carry

        lax.fori_loop(0, steps, step, 0)
        plsc.subcore_barrier()
        pltpu.sync_copy(acc.at[pl.ds(s * rps, rps)],
                        out_hbm.at[c].at[pl.ds(s * rps, rps)])

    return k(h, n0r, n1r, zrows)


def kernel(node_attributes, edge_attributes, W_node, b_node, W_edge, b_edge,
           W_att, b_att, edge_node_indices):
    n, d = node_attributes.shape
    e = edge_node_indices.shape[1]

    steps = -(-e // (NW * CH))
    e_pad = NW * steps * CH
    n_pad = -(-(n + 1) // (NS * 8)) * (NS * 8)

    n0 = edge_node_indices[0]
    n1 = edge_node_indices[1]
    pad = e_pad - e
    n0r = jnp.concatenate([n0, jnp.full((pad,), n, jnp.int32)]).reshape(-1, CH)
    n1r = jnp.concatenate([n1, jnp.zeros((pad,), jnp.int32)]).reshape(-1, CH)
    zrows = jnp.zeros((n_pad // NS, d), jnp.float32)

    h = _node_linear(node_attributes, W_node, b_node)
    partials = _scatter_partials(h, n0r, n1r, zrows, n_pad, steps)
    return _combine(partials[0, :n], partials[1, :n])


# speedup vs baseline: 3.9927x; 2.7328x over previous
import functools

import jax
import jax.numpy as jnp
from jax import lax
from jax.experimental import pallas as pl
from jax.experimental.pallas import tpu as pltpu
from jax.experimental.pallas import tpu_sc as plsc

NC = 2
NS = 16
NW = NC * NS
CH = 128


def _node_linear(x, w, b):
    n, d = x.shape
    br = 1000
    grid = (n // br,)

    def body(x_ref, w_ref, b_ref, o_ref):
        o_ref[...] = lax.dot_general(
            x_ref[...], w_ref[...], (((1,), (1,)), ((), ())),
            preferred_element_type=jnp.float32) + b_ref[...]

    return pl.pallas_call(
        body,
        grid=grid,
        in_specs=[
            pl.BlockSpec((br, d), lambda i: (i, 0)),
            pl.BlockSpec((d, d), lambda i: (0, 0)),
            pl.BlockSpec((1, d), lambda i: (0, 0)),
        ],
        out_specs=pl.BlockSpec((br, d), lambda i: (i, 0)),
        out_shape=jax.ShapeDtypeStruct((n, d), jnp.float32),
    )(x, w, b.reshape(1, d))


def _combine(p0, p1):
    n, d = p0.shape
    br = 1000
    grid = (n // br,)

    def body(a_ref, b_ref, o_ref):
        o_ref[...] = a_ref[...] + b_ref[...]

    return pl.pallas_call(
        body,
        grid=grid,
        in_specs=[
            pl.BlockSpec((br, d), lambda i: (i, 0)),
            pl.BlockSpec((br, d), lambda i: (i, 0)),
        ],
        out_specs=pl.BlockSpec((br, d), lambda i: (i, 0)),
        out_shape=jax.ShapeDtypeStruct((n, d), jnp.float32),
    )(p0, p1)


NBUF = 2


def _scatter_partials(h, n0r, n1r, zrows, n_pad, steps):
    _, d = h.shape
    rps = n_pad // NS
    mesh = plsc.VectorSubcoreMesh(core_axis_name="c", subcore_axis_name="s")

    @functools.partial(
        pl.kernel,
        out_type=jax.ShapeDtypeStruct((NC, n_pad, d), jnp.float32),
        mesh=mesh,
        scratch_types=[
            pltpu.VMEM((steps, CH), jnp.int32),
            pltpu.VMEM((NBUF, CH), jnp.int32),
            [pltpu.VMEM((CH, d), jnp.float32)] * NBUF,
            pltpu.VMEM_SHARED((n_pad, d), jnp.float32),
            [pltpu.SemaphoreType.DMA] * NBUF,
            [pltpu.SemaphoreType.DMA] * NBUF,
        ],
    )
    def k(h_hbm, n0_hbm, n1_hbm, z_hbm, out_hbm, i1a, i0r, rows, acc,
          rsems, isems):
        c = lax.axis_index("c")
        s = lax.axis_index("s")
        base = (c * NS + s) * steps
        pltpu.sync_copy(n1_hbm.at[pl.ds(base, steps)], i1a)
        pltpu.sync_copy(z_hbm, acc.at[pl.ds(s * rps, rps)])
        plsc.subcore_barrier()

        for b in range(NBUF):
            pltpu.async_copy(n0_hbm.at[base + b], i0r.at[b], isems[b])
            pltpu.async_copy(h_hbm.at[i1a.at[b]], rows[b], rsems[b])

        def step(i, carry):
            for b in range(NBUF):
                ch = i * NBUF + b
                pltpu.make_async_copy(
                    h_hbm.at[pl.ds(0, CH)], rows[b], rsems[b]).wait()
                pltpu.make_async_copy(
                    n0_hbm.at[0], i0r.at[b], isems[b]).wait()
                pltpu.sync_copy(rows[b], acc.at[i0r.at[b]], add=True)

                @pl.when(ch + NBUF < steps)
                def _():
                    pltpu.async_copy(
                        n0_hbm.at[base + ch + NBUF], i0r.at[b], isems[b])
                    pltpu.async_copy(
                        h_hbm.at[i1a.at[ch + NBUF]], rows[b], rsems[b])
            return carry

        lax.fori_loop(0, steps // NBUF, step, 0)
        plsc.subcore_barrier()
        pltpu.sync_copy(acc.at[pl.ds(s * rps, rps)],
                        out_hbm.at[c].at[pl.ds(s * rps, rps)])

    return k(h, n0r, n1r, zrows)


def kernel(node_attributes, edge_attributes, W_node, b_node, W_edge, b_edge,
           W_att, b_att, edge_node_indices):
    n, d = node_attributes.shape
    e = edge_node_indices.shape[1]

    steps = -(-e // (NW * CH * 8)) * 8
    e_pad = NW * steps * CH
    n_pad = -(-(n + 1) // (NS * 8)) * (NS * 8)

    n0 = edge_node_indices[0]
    n1 = edge_node_indices[1]
    pad = e_pad - e
    pad_n0 = n + jax.lax.rem(jnp.arange(pad, dtype=jnp.int32),
                             jnp.int32(n_pad - n))
    pad_n1 = jax.lax.rem(jnp.arange(pad, dtype=jnp.int32), jnp.int32(n))
    n0r = jnp.concatenate([n0, pad_n0]).reshape(-1, CH)
    n1r = jnp.concatenate([n1, pad_n1]).reshape(-1, CH)
    zrows = jnp.zeros((n_pad // NS, d), jnp.float32)

    h = _node_linear(node_attributes, W_node, b_node)
    partials = _scatter_partials(h, n0r, n1r, zrows, n_pad, steps)
    return _combine(partials[0, :n], partials[1, :n])


# speedup vs baseline: 4.1510x; 1.0396x over previous
import functools

import jax
import jax.numpy as jnp
from jax import lax
from jax.experimental import pallas as pl
from jax.experimental.pallas import tpu as pltpu
from jax.experimental.pallas import tpu_sc as plsc

NC = 2
NS = 16
NW = NC * NS
CH = 128


def _node_linear(x, w, b):
    n, d = x.shape
    br = 1000
    grid = (n // br,)

    def body(x_ref, w_ref, b_ref, o_ref):
        o_ref[...] = lax.dot_general(
            x_ref[...], w_ref[...], (((1,), (1,)), ((), ())),
            preferred_element_type=jnp.float32) + b_ref[...]

    return pl.pallas_call(
        body,
        grid=grid,
        in_specs=[
            pl.BlockSpec((br, d), lambda i: (i, 0)),
            pl.BlockSpec((d, d), lambda i: (0, 0)),
            pl.BlockSpec((1, d), lambda i: (0, 0)),
        ],
        out_specs=pl.BlockSpec((br, d), lambda i: (i, 0)),
        out_shape=jax.ShapeDtypeStruct((n, d), jnp.float32),
    )(x, w, b.reshape(1, d))


def _combine(partials, n):
    _, _, d = partials.shape
    br = 1000
    grid = (n // br,)

    def body(a_ref, b_ref, o_ref):
        o_ref[...] = a_ref[0] + b_ref[0]

    return pl.pallas_call(
        body,
        grid=grid,
        in_specs=[
            pl.BlockSpec((1, br, d), lambda i: (0, i, 0)),
            pl.BlockSpec((1, br, d), lambda i: (1, i, 0)),
        ],
        out_specs=pl.BlockSpec((br, d), lambda i: (i, 0)),
        out_shape=jax.ShapeDtypeStruct((n, d), jnp.float32),
    )(partials, partials)


NBUF = 2


def _scatter_partials(h, n0r, n1r, zrows, n_pad, steps):
    _, d = h.shape
    rps = n_pad // NS
    mesh = plsc.VectorSubcoreMesh(core_axis_name="c", subcore_axis_name="s")

    @functools.partial(
        pl.kernel,
        out_type=jax.ShapeDtypeStruct((NC, n_pad, d), jnp.float32),
        mesh=mesh,
        scratch_types=[
            pltpu.VMEM((steps, CH), jnp.int32),
            pltpu.VMEM((NBUF, CH), jnp.int32),
            [pltpu.VMEM((CH, d), jnp.float32)] * NBUF,
            pltpu.VMEM_SHARED((n_pad, d), jnp.float32),
            [pltpu.SemaphoreType.DMA] * NBUF,
            [pltpu.SemaphoreType.DMA] * NBUF,
        ],
    )
    def k(h_hbm, n0_hbm, n1_hbm, z_hbm, out_hbm, i1a, i0r, rows, acc,
          rsems, isems):
        c = lax.axis_index("c")
        s = lax.axis_index("s")
        base = (c * NS + s) * steps
        pltpu.sync_copy(n1_hbm.at[pl.ds(base, steps)], i1a)
        pltpu.sync_copy(z_hbm, acc.at[pl.ds(s * rps, rps)])
        plsc.subcore_barrier()

        for b in range(NBUF):
            pltpu.async_copy(n0_hbm.at[base + b], i0r.at[b], isems[b])
            pltpu.async_copy(h_hbm.at[i1a.at[b]], rows[b], rsems[b])

        def step(i, carry):
            for b in range(NBUF):
                ch = i * NBUF + b
                pltpu.make_async_copy(
                    h_hbm.at[pl.ds(0, CH)], rows[b], rsems[b]).wait()
                pltpu.make_async_copy(
                    n0_hbm.at[0], i0r.at[b], isems[b]).wait()
                pltpu.sync_copy(rows[b], acc.at[i0r.at[b]], add=True)

                @pl.when(ch + NBUF < steps)
                def _():
                    pltpu.async_copy(
                        n0_hbm.at[base + ch + NBUF], i0r.at[b], isems[b])
                    pltpu.async_copy(
                        h_hbm.at[i1a.at[ch + NBUF]], rows[b], rsems[b])
            return carry

        lax.fori_loop(0, steps // NBUF, step, 0)
        plsc.subcore_barrier()
        pltpu.sync_copy(acc.at[pl.ds(s * rps, rps)],
                        out_hbm.at[c].at[pl.ds(s * rps, rps)])

    return k(h, n0r, n1r, zrows)


def kernel(node_attributes, edge_attributes, W_node, b_node, W_edge, b_edge,
           W_att, b_att, edge_node_indices):
    n, d = node_attributes.shape
    e = edge_node_indices.shape[1]

    steps = -(-e // (NW * CH * 8)) * 8
    e_pad = NW * steps * CH
    n_pad = -(-(n + 1) // (NS * 8)) * (NS * 8)

    n0 = edge_node_indices[0]
    n1 = edge_node_indices[1]
    pad = e_pad - e
    pad_n0 = n + jax.lax.rem(jnp.arange(pad, dtype=jnp.int32),
                             jnp.int32(n_pad - n))
    pad_n1 = jax.lax.rem(jnp.arange(pad, dtype=jnp.int32), jnp.int32(n))
    n0r = jnp.concatenate([n0, pad_n0]).reshape(-1, CH)
    n1r = jnp.concatenate([n1, pad_n1]).reshape(-1, CH)
    zrows = jnp.zeros((n_pad // NS, d), jnp.float32)

    h = _node_linear(node_attributes, W_node, b_node)
    partials = _scatter_partials(h, n0r, n1r, zrows, n_pad, steps)
    return _combine(partials, n)


# speedup vs baseline: 4.1816x; 1.0074x over previous
import functools

import jax
import jax.numpy as jnp
from jax import lax
from jax.experimental import pallas as pl
from jax.experimental.pallas import tpu as pltpu
from jax.experimental.pallas import tpu_sc as plsc

NC = 2
NS = 16
NW = NC * NS
CH = 64


def _node_linear(x, w, b):
    n, d = x.shape
    br = 1000
    grid = (n // br,)

    def body(x_ref, w_ref, b_ref, o_ref):
        o_ref[...] = lax.dot_general(
            x_ref[...], w_ref[...], (((1,), (1,)), ((), ())),
            preferred_element_type=jnp.float32) + b_ref[...]

    return pl.pallas_call(
        body,
        grid=grid,
        in_specs=[
            pl.BlockSpec((br, d), lambda i: (i, 0)),
            pl.BlockSpec((d, d), lambda i: (0, 0)),
            pl.BlockSpec((1, d), lambda i: (0, 0)),
        ],
        out_specs=pl.BlockSpec((br, d), lambda i: (i, 0)),
        out_shape=jax.ShapeDtypeStruct((n, d), jnp.float32),
    )(x, w, b.reshape(1, d))


def _combine(partials, n):
    _, _, d = partials.shape
    br = 1000
    grid = (n // br,)

    def body(a_ref, b_ref, o_ref):
        o_ref[...] = a_ref[0] + b_ref[0]

    return pl.pallas_call(
        body,
        grid=grid,
        in_specs=[
            pl.BlockSpec((1, br, d), lambda i: (0, i, 0)),
            pl.BlockSpec((1, br, d), lambda i: (1, i, 0)),
        ],
        out_specs=pl.BlockSpec((br, d), lambda i: (i, 0)),
        out_shape=jax.ShapeDtypeStruct((n, d), jnp.float32),
    )(partials, partials)


NBUF = 4


def _scatter_partials(h, n0r, n1r, zrows, n_pad, steps):
    _, d = h.shape
    rps = n_pad // NS
    mesh = plsc.VectorSubcoreMesh(core_axis_name="c", subcore_axis_name="s")

    @functools.partial(
        pl.kernel,
        out_type=jax.ShapeDtypeStruct((NC, n_pad, d), jnp.float32),
        mesh=mesh,
        scratch_types=[
            pltpu.VMEM((NBUF, CH), jnp.int32),
            pltpu.VMEM((NBUF, CH), jnp.int32),
            [pltpu.VMEM((CH, d), jnp.float32)] * NBUF,
            pltpu.VMEM_SHARED((n_pad, d), jnp.float32),
            [pltpu.SemaphoreType.DMA] * NBUF,
            [pltpu.SemaphoreType.DMA] * NBUF,
            [pltpu.SemaphoreType.DMA] * NBUF,
        ],
    )
    def k(h_hbm, n0_hbm, n1_hbm, z_hbm, out_hbm, i1r, i0r, rows, acc,
          rsems, isems, jsems):
        c = lax.axis_index("c")
        s = lax.axis_index("s")
        base = (c * NS + s) * steps
        pltpu.sync_copy(z_hbm, acc.at[pl.ds(s * rps, rps)])
        plsc.subcore_barrier()

        for b in range(NBUF):
            pltpu.async_copy(n0_hbm.at[base + b], i0r.at[b], isems[b])
            pltpu.async_copy(n1_hbm.at[base + b], i1r.at[b], jsems[b])
        for b in range(NBUF):
            pltpu.make_async_copy(n1_hbm.at[0], i1r.at[b], jsems[b]).wait()
            pltpu.async_copy(h_hbm.at[i1r.at[b]], rows[b], rsems[b])

        def step(i, carry):
            for b in range(NBUF):
                ch = i * NBUF + b
                pltpu.make_async_copy(
                    h_hbm.at[pl.ds(0, CH)], rows[b], rsems[b]).wait()

                @pl.when(ch + NBUF < steps)
                def _():
                    pltpu.async_copy(
                        n1_hbm.at[base + ch + NBUF], i1r.at[b], jsems[b])
                pltpu.make_async_copy(
                    n0_hbm.at[0], i0r.at[b], isems[b]).wait()
                pltpu.sync_copy(rows[b], acc.at[i0r.at[b]], add=True)

                @pl.when(ch + NBUF < steps)
                def _():
                    pltpu.async_copy(
                        n0_hbm.at[base + ch + NBUF], i0r.at[b], isems[b])
                    pltpu.make_async_copy(
                        n1_hbm.at[0], i1r.at[b], jsems[b]).wait()
                    pltpu.async_copy(
                        h_hbm.at[i1r.at[b]], rows[b], rsems[b])
            return carry

        lax.fori_loop(0, steps // NBUF, step, 0)
        plsc.subcore_barrier()
        pltpu.sync_copy(acc.at[pl.ds(s * rps, rps)],
                        out_hbm.at[c].at[pl.ds(s * rps, rps)])

    return k(h, n0r, n1r, zrows)


def kernel(node_attributes, edge_attributes, W_node, b_node, W_edge, b_edge,
           W_att, b_att, edge_node_indices):
    n, d = node_attributes.shape
    e = edge_node_indices.shape[1]

    steps = -(-e // (NW * CH * 8)) * 8
    e_pad = NW * steps * CH
    n_pad = -(-(n + 1) // (NS * 8)) * (NS * 8)

    n0 = edge_node_indices[0]
    n1 = edge_node_indices[1]
    pad = e_pad - e
    pad_n0 = n + jax.lax.rem(jnp.arange(pad, dtype=jnp.int32),
                             jnp.int32(n_pad - n))
    pad_n1 = jax.lax.rem(jnp.arange(pad, dtype=jnp.int32), jnp.int32(n))
    n0r = jnp.concatenate([n0, pad_n0]).reshape(-1, CH)
    n1r = jnp.concatenate([n1, pad_n1]).reshape(-1, CH)
    zrows = jnp.zeros((n_pad // NS, d), jnp.float32)

    h = _node_linear(node_attributes, W_node, b_node)
    partials = _scatter_partials(h, n0r, n1r, zrows, n_pad, steps)
    return _combine(partials, n)


# speedup vs baseline: 4.2442x; 1.0150x over previous
import functools

import jax
import jax.numpy as jnp
from jax import lax
from jax.experimental import pallas as pl
from jax.experimental.pallas import tpu as pltpu
from jax.experimental.pallas import tpu_sc as plsc

NC = 2
NS = 16
NW = NC * NS
CH = 64


def _node_linear(x, w, b):
    n, d = x.shape
    br = 1000
    grid = (n // br,)

    def body(x_ref, w_ref, b_ref, o_ref):
        o_ref[...] = lax.dot_general(
            x_ref[...], w_ref[...], (((1,), (1,)), ((), ())),
            preferred_element_type=jnp.float32) + b_ref[...]

    return pl.pallas_call(
        body,
        grid=grid,
        in_specs=[
            pl.BlockSpec((br, d), lambda i: (i, 0)),
            pl.BlockSpec((d, d), lambda i: (0, 0)),
            pl.BlockSpec((1, d), lambda i: (0, 0)),
        ],
        out_specs=pl.BlockSpec((br, d), lambda i: (i, 0)),
        out_shape=jax.ShapeDtypeStruct((n, d), jnp.float32),
    )(x, w, b.reshape(1, d))


def _combine(partials, n):
    _, _, d = partials.shape
    br = 1000
    grid = (n // br,)

    def body(a_ref, b_ref, o_ref):
        o_ref[...] = a_ref[0] + b_ref[0]

    return pl.pallas_call(
        body,
        grid=grid,
        in_specs=[
            pl.BlockSpec((1, br, d), lambda i: (0, i, 0)),
            pl.BlockSpec((1, br, d), lambda i: (1, i, 0)),
        ],
        out_specs=pl.BlockSpec((br, d), lambda i: (i, 0)),
        out_shape=jax.ShapeDtypeStruct((n, d), jnp.float32),
    )(partials, partials)


NBUF = 4


def _scatter_partials(h, n0r, n1r, zrows, n_pad, steps):
    _, d = h.shape
    rps = n_pad // NS
    mesh = plsc.VectorSubcoreMesh(core_axis_name="c", subcore_axis_name="s")

    @functools.partial(
        pl.kernel,
        out_type=jax.ShapeDtypeStruct((NC, n_pad, d), jnp.float32),
        mesh=mesh,
        scratch_types=[
            pltpu.VMEM((NBUF, CH), jnp.int32),
            pltpu.VMEM((NBUF, CH), jnp.int32),
            [pltpu.VMEM((CH, d), jnp.float32)] * NBUF,
            pltpu.VMEM_SHARED((n_pad, d), jnp.float32),
            [pltpu.SemaphoreType.DMA] * NBUF,
            [pltpu.SemaphoreType.DMA] * NBUF,
            [pltpu.SemaphoreType.DMA] * NBUF,
        ],
    )
    def k(h_hbm, n0_hbm, n1_hbm, z_hbm, out_hbm, i1r, i0r, rows, acc,
          rsems, isems, jsems):
        c = lax.axis_index("c")
        s = lax.axis_index("s")
        base = (c * NS + s) * steps
        total = n0_hbm.shape[0]
        nch = jnp.clip(total - base, 0, steps)
        pltpu.sync_copy(z_hbm, acc.at[pl.ds(s * rps, rps)])
        plsc.subcore_barrier()

        for b in range(NBUF):
            @pl.when(b < nch)
            def _():
                pltpu.async_copy(n0_hbm.at[base + b], i0r.at[b], isems[b])
                pltpu.async_copy(n1_hbm.at[base + b], i1r.at[b], jsems[b])
        for b in range(NBUF):
            @pl.when(b < nch)
            def _():
                pltpu.make_async_copy(
                    n1_hbm.at[0], i1r.at[b], jsems[b]).wait()
                pltpu.async_copy(h_hbm.at[i1r.at[b]], rows[b], rsems[b])

        def step(i, carry):
            for b in range(NBUF):
                ch = i * NBUF + b

                @pl.when(ch < nch)
                def _():
                    pltpu.make_async_copy(
                        h_hbm.at[pl.ds(0, CH)], rows[b], rsems[b]).wait()

                    @pl.when(ch + NBUF < nch)
                    def _():
                        pltpu.async_copy(
                            n1_hbm.at[base + ch + NBUF], i1r.at[b], jsems[b])
                    pltpu.make_async_copy(
                        n0_hbm.at[0], i0r.at[b], isems[b]).wait()
                    pltpu.sync_copy(rows[b], acc.at[i0r.at[b]], add=True)

                    @pl.when(ch + NBUF < nch)
                    def _():
                        pltpu.async_copy(
                            n0_hbm.at[base + ch + NBUF], i0r.at[b], isems[b])
                        pltpu.make_async_copy(
                            n1_hbm.at[0], i1r.at[b], jsems[b]).wait()
                        pltpu.async_copy(
                            h_hbm.at[i1r.at[b]], rows[b], rsems[b])
            return carry

        lax.fori_loop(0, (nch + NBUF - 1) // NBUF, step, 0)
        plsc.subcore_barrier()
        pltpu.sync_copy(acc.at[pl.ds(s * rps, rps)],
                        out_hbm.at[c].at[pl.ds(s * rps, rps)])

    return k(h, n0r, n1r, zrows)


def kernel(node_attributes, edge_attributes, W_node, b_node, W_edge, b_edge,
           W_att, b_att, edge_node_indices):
    n, d = node_attributes.shape
    e = edge_node_indices.shape[1]

    e_pad = -(-e // CH) * CH
    steps = -(-(e_pad // CH) // NW)
    n_pad = -(-(n + 1) // (NS * 8)) * (NS * 8)

    n0 = edge_node_indices[0]
    n1 = edge_node_indices[1]
    pad = e_pad - e
    if pad:
        pad_n0 = n + jax.lax.rem(jnp.arange(pad, dtype=jnp.int32),
                                 jnp.int32(n_pad - n))
        pad_n1 = jax.lax.rem(jnp.arange(pad, dtype=jnp.int32), jnp.int32(n))
        n0 = jnp.concatenate([n0, pad_n0])
        n1 = jnp.concatenate([n1, pad_n1])
    n0r = n0.reshape(-1, CH)
    n1r = n1.reshape(-1, CH)
    zrows = jnp.zeros((n_pad // NS, d), jnp.float32)

    h = _node_linear(node_attributes, W_node, b_node)
    partials = _scatter_partials(h, n0r, n1r, zrows, n_pad, steps)
    return _combine(partials, n)
